# baseline (speedup 1.0000x reference)
import functools

import jax
import jax.numpy as jnp
from jax.experimental import pallas as pl
from jax.experimental.pallas import tpu as pltpu

_CHUNKS = (128, 256, 512, 512, 512, 128)


def _stream_kernel(x_hbm, w_ref, b_ref, o_ref, x_vmem, sems):
    n = len(_CHUNKS)
    offs = [sum(_CHUNKS[:i]) for i in range(n)]

    for i in range(n):
        rows = pl.ds(offs[i], _CHUNKS[i])
        pltpu.make_async_copy(x_hbm.at[rows, :], x_vmem.at[rows, :],
                              sems.at[i]).start()

    for i in range(n):
        rows = pl.ds(offs[i], _CHUNKS[i])
        pltpu.make_async_copy(x_hbm.at[rows, :], x_vmem.at[rows, :],
                              sems.at[i]).wait()
        o_ref[rows, :] = jnp.dot(x_vmem[rows, :], w_ref[...],
                                 preferred_element_type=jnp.float32) + b_ref[...]


@functools.partial(jax.jit, static_argnames=("output_size",))
def _forward(bow_vec, w_p, b_p, *, output_size):
    B, F = bow_vec.shape
    F_pad, O_pad = w_p.shape

    out = pl.pallas_call(
        _stream_kernel,
        out_shape=jax.ShapeDtypeStruct((B, O_pad), jnp.float32),
        grid=(1,),
        in_specs=[
            pl.BlockSpec(memory_space=pl.ANY),
            pl.BlockSpec((F_pad, O_pad), lambda i: (0, 0)),
            pl.BlockSpec((1, O_pad), lambda i: (0, 0)),
        ],
        out_specs=pl.BlockSpec((B, O_pad), lambda i: (0, 0)),
        scratch_shapes=[
            pltpu.VMEM((B, F_pad), jnp.float32),
            pltpu.SemaphoreType.DMA((len(_CHUNKS),)),
        ],
        compiler_params=pltpu.CompilerParams(
            dimension_semantics=("arbitrary",),
            vmem_limit_bytes=56 * 1024 * 1024,
        ),
    )(bow_vec, w_p, b_p)
    return out[:, :output_size]


def kernel(bow_vec, w_p, b_p):
    return _forward(bow_vec, w_p, b_p, output_size=100)


# speedup vs baseline: 1.1113x; 1.1113x over previous
import functools

import jax
import jax.numpy as jnp
from jax.experimental import pallas as pl
from jax.experimental.pallas import tpu as pltpu


def _linear_kernel(x_ref, w_ref, b_ref, o_ref):
    o_ref[...] = (
        jnp.dot(x_ref[...], w_ref[...], preferred_element_type=jnp.float32)
        + b_ref[...]
    ).astype(o_ref.dtype)


@functools.partial(jax.jit, static_argnames=("output_size", "tm"))
def _forward(bow_vec, w_p, b_p, *, output_size, tm):
    B, F = bow_vec.shape
    F_pad, O_pad = w_p.shape

    out = pl.pallas_call(
        _linear_kernel,
        out_shape=jax.ShapeDtypeStruct((B, O_pad), jnp.float32),
        grid=(B // tm,),
        in_specs=[
            pl.BlockSpec((tm, F_pad), lambda i: (i, 0)),
            pl.BlockSpec((F_pad, O_pad), lambda i: (0, 0)),
            pl.BlockSpec((1, O_pad), lambda i: (0, 0)),
        ],
        out_specs=pl.BlockSpec((tm, O_pad), lambda i: (i, 0)),
        compiler_params=pltpu.CompilerParams(
            dimension_semantics=("arbitrary",),
            vmem_limit_bytes=56 * 1024 * 1024,
        ),
    )(bow_vec, w_p, b_p)
    return out[:, :output_size]


def kernel(bow_vec, w_p, b_p):
    return _forward(bow_vec, w_p, b_p, output_size=100, tm=1024)


# speedup vs baseline: 1.2328x; 1.1093x over previous
import functools

import jax
import jax.numpy as jnp
from jax.experimental import pallas as pl
from jax.experimental.pallas import tpu as pltpu


def _linear_kernel(x_ref, w_ref, b_ref, o_ref):
    acc = jnp.dot(x_ref[...], w_ref[...],
                  preferred_element_type=jnp.float32) + b_ref[...]
    o_ref[...] = acc[:, : o_ref.shape[1]]


@functools.partial(jax.jit, static_argnames=("output_size", "tm"))
def _forward(bow_vec, w_p, b_p, *, output_size, tm):
    B, F = bow_vec.shape
    F_pad, O_pad = w_p.shape

    return pl.pallas_call(
        _linear_kernel,
        out_shape=jax.ShapeDtypeStruct((B, output_size), jnp.float32),
        grid=(B // tm,),
        in_specs=[
            pl.BlockSpec((tm, F_pad), lambda i: (i, 0)),
            pl.BlockSpec((F_pad, O_pad), lambda i: (0, 0)),
            pl.BlockSpec((1, O_pad), lambda i: (0, 0)),
        ],
        out_specs=pl.BlockSpec((tm, output_size), lambda i: (i, 0)),
        compiler_params=pltpu.CompilerParams(
            dimension_semantics=("arbitrary",),
            vmem_limit_bytes=48 * 1024 * 1024,
        ),
    )(bow_vec, w_p, b_p)


def kernel(bow_vec, w_p, b_p):
    return _forward(bow_vec, w_p, b_p, output_size=100, tm=512)
